# baseline (speedup 1.0000x reference)
import functools

import jax
import jax.numpy as jnp
from jax import lax
from jax.experimental import pallas as pl
from jax.experimental.pallas import tpu as pltpu
from jax.experimental.pallas import tpu_sc as plsc


@functools.cache
def _gather_call(B, E, dtype):
    info = plsc.get_sparse_core_info()
    NC, NS = info.num_cores, info.num_subcores
    NW = NC * NS
    b_per_w = B // NW
    mesh = plsc.VectorSubcoreMesh(core_axis_name="c", subcore_axis_name="s")

    @functools.partial(
        pl.kernel,
        mesh=mesh,
        out_type=[
            jax.ShapeDtypeStruct((B, E), dtype),
            jax.ShapeDtypeStruct((B, E), dtype),
        ],
        scratch_types=[
            pltpu.VMEM((b_per_w,), jnp.int32),
            pltpu.VMEM((b_per_w, E), dtype),
            pltpu.VMEM((b_per_w,), jnp.int32),
            pltpu.VMEM((b_per_w, E), dtype),
            pltpu.SemaphoreType.DMA,
            pltpu.SemaphoreType.DMA,
        ],
        compiler_params=pltpu.CompilerParams(use_tc_tiling_on_sc=False),
    )
    def gather_k(uids_hbm, iids_hbm, utab_hbm, itab_hbm, u_out, v_out,
                 uidx_v, urows_v, iidx_v, vrows_v, usem, isem):
        wid = lax.axis_index("s") * NC + lax.axis_index("c")
        base = wid * b_per_w
        pltpu.sync_copy(uids_hbm.at[pl.ds(base, b_per_w)], uidx_v)
        pltpu.sync_copy(iids_hbm.at[pl.ds(base, b_per_w)], iidx_v)
        ucp = pltpu.async_copy(utab_hbm.at[uidx_v], urows_v, usem)
        icp = pltpu.async_copy(itab_hbm.at[iidx_v], vrows_v, isem)
        ucp.wait()
        icp.wait()
        pltpu.sync_copy(urows_v, u_out.at[pl.ds(base, b_per_w)])
        pltpu.sync_copy(vrows_v, v_out.at[pl.ds(base, b_per_w)])

    return gather_k


def _mlp_body(u_ref, v_ref, w1u_ref, w1v_ref, b1_ref, w2_ref, b2_ref,
              out_ref):
    h = jnp.dot(u_ref[...], w1u_ref[...], preferred_element_type=jnp.float32)
    h = h + jnp.dot(v_ref[...], w1v_ref[...],
                    preferred_element_type=jnp.float32)
    h = jnp.maximum(h + b1_ref[...], 0.0)
    out_ref[...] = (
        jnp.dot(h, w2_ref[...], preferred_element_type=jnp.float32)
        + b2_ref[0, 0]
    )


@functools.cache
def _mlp_call(B, E, H, BB):
    grid = (B // BB,)
    return pl.pallas_call(
        _mlp_body,
        grid=grid,
        in_specs=[
            pl.BlockSpec((BB, E), lambda i: (i, 0)),
            pl.BlockSpec((BB, E), lambda i: (i, 0)),
            pl.BlockSpec((E, H), lambda i: (0, 0)),
            pl.BlockSpec((E, H), lambda i: (0, 0)),
            pl.BlockSpec((1, H), lambda i: (0, 0)),
            pl.BlockSpec((H, 1), lambda i: (0, 0)),
            pl.BlockSpec((1, 1), lambda i: (0, 0)),
        ],
        out_specs=pl.BlockSpec((BB, 1), lambda i: (i, 0)),
        out_shape=jax.ShapeDtypeStruct((B, 1), jnp.float32),
    )


def kernel(user_ids, item_ids, user_table, item_table, W1, b1, W2, b2):
    B = user_ids.shape[0]
    E = user_table.shape[1]
    H = W1.shape[0]

    u_g, v_g = _gather_call(B, E, user_table.dtype)(
        user_ids, item_ids, user_table, item_table)

    w1u = W1[:, :E].T
    w1v = W1[:, E:].T
    return _mlp_call(B, E, H, 2048)(
        u_g, v_g, w1u, w1v, b1.reshape(1, H), W2.T, b2.reshape(1, 1))


# speedup vs baseline: 3.3574x; 3.3574x over previous
import functools

import jax
import jax.numpy as jnp
from jax import lax
from jax.experimental import pallas as pl
from jax.experimental.pallas import tpu as pltpu
from jax.experimental.pallas import tpu_sc as plsc

_LANES = 128


@functools.cache
def _gather_call(B, E, dtype):
    info = plsc.get_sparse_core_info()
    NC, NS, L = info.num_cores, info.num_subcores, info.num_lanes
    NW = NC * NS
    half = NW // 2
    b_per_w = B // half
    n_groups = b_per_w // L
    mesh = plsc.VectorSubcoreMesh(core_axis_name="c", subcore_axis_name="s")

    @functools.partial(
        pl.kernel,
        mesh=mesh,
        out_type=[
            jax.ShapeDtypeStruct((E, B), dtype),
            jax.ShapeDtypeStruct((E, B), dtype),
        ],
        scratch_types=[
            pltpu.VMEM((b_per_w,), jnp.int32),
            pltpu.VMEM((L, E, _LANES), dtype),
            pltpu.VMEM((E, b_per_w), dtype),
            pltpu.SemaphoreType.DMA,
        ],
        compiler_params=pltpu.CompilerParams(needs_layout_passes=False),
    )
    def gather_k(uids_hbm, iids_hbm, utabT_hbm, itabT_hbm, uT_out, vT_out,
                 idx_v, blocks_v, outT_v, sem):
        wid = lax.axis_index("s") * NC + lax.axis_index("c")
        base = lax.rem(wid, half) * b_per_w
        e_lo = jnp.arange(L, dtype=jnp.int32)
        e_hi = e_lo + L

        def do_table(ids_hbm, tabT_hbm, out_hbm):
            pltpu.sync_copy(ids_hbm.at[pl.ds(base, b_per_w)], idx_v)

            def group(g, carry):
                idxvec = idx_v[pl.ds(g * L, L)]
                for j in range(L):
                    tc = idxvec[j] // _LANES
                    off = pl.multiple_of(tc * _LANES, _LANES)
                    pltpu.async_copy(
                        tabT_hbm.at[:, pl.ds(off, _LANES)],
                        blocks_v.at[j], sem)
                for j in range(L):
                    pltpu.make_async_copy(
                        tabT_hbm.at[:, pl.ds(0, _LANES)],
                        blocks_v.at[j], sem).wait()
                for j in range(L):
                    ln = jnp.full((L,), lax.rem(idxvec[j], _LANES),
                                  dtype=jnp.int32)
                    k = jnp.full((L,), g * L + j, dtype=jnp.int32)
                    lo = plsc.load_gather(blocks_v.at[j], [e_lo, ln])
                    hi = plsc.load_gather(blocks_v.at[j], [e_hi, ln])
                    plsc.store_scatter(outT_v, [e_lo, k], lo)
                    plsc.store_scatter(outT_v, [e_hi, k], hi)
                return carry

            lax.fori_loop(0, n_groups, group, 0)
            pltpu.sync_copy(outT_v, out_hbm.at[:, pl.ds(base, b_per_w)])

        @pl.when(wid < half)
        def _():
            do_table(uids_hbm, utabT_hbm, uT_out)

        @pl.when(wid >= half)
        def _():
            do_table(iids_hbm, itabT_hbm, vT_out)

    return gather_k


def _mlp_body(uT_ref, vT_ref, w1u_ref, w1v_ref, b1_ref, w2_ref, b2_ref,
              out_ref):
    hT = jnp.dot(w1u_ref[...], uT_ref[...], preferred_element_type=jnp.float32)
    hT = hT + jnp.dot(w1v_ref[...], vT_ref[...],
                      preferred_element_type=jnp.float32)
    hT = jnp.maximum(hT + b1_ref[...], 0.0)
    out_ref[...] = (
        jnp.dot(w2_ref[...], hT, preferred_element_type=jnp.float32)
        + b2_ref[0, 0]
    )


@functools.cache
def _mlp_call(B, E, H, BB):
    grid = (B // BB,)
    return pl.pallas_call(
        _mlp_body,
        grid=grid,
        in_specs=[
            pl.BlockSpec((E, BB), lambda i: (0, i)),
            pl.BlockSpec((E, BB), lambda i: (0, i)),
            pl.BlockSpec((H, E), lambda i: (0, 0)),
            pl.BlockSpec((H, E), lambda i: (0, 0)),
            pl.BlockSpec((H, 1), lambda i: (0, 0)),
            pl.BlockSpec((1, H), lambda i: (0, 0)),
            pl.BlockSpec((1, 1), lambda i: (0, 0)),
        ],
        out_specs=pl.BlockSpec((1, BB), lambda i: (0, i)),
        out_shape=jax.ShapeDtypeStruct((1, B), jnp.float32),
    )


def kernel(user_ids, item_ids, user_table, item_table, W1, b1, W2, b2):
    B = user_ids.shape[0]
    E = user_table.shape[1]
    H = W1.shape[0]

    uT, vT = _gather_call(B, E, user_table.dtype)(
        user_ids, item_ids, user_table.T, item_table.T)

    w1u = W1[:, :E]
    w1v = W1[:, E:]
    logitsT = _mlp_call(B, E, H, 2048)(
        uT, vT, w1u, w1v, b1.reshape(H, 1), W2, b2.reshape(1, 1))
    return logitsT.reshape(B, 1)


# speedup vs baseline: 3.5577x; 1.0596x over previous
import functools

import jax
import jax.numpy as jnp
from jax import lax
from jax.experimental import pallas as pl
from jax.experimental.pallas import tpu as pltpu
from jax.experimental.pallas import tpu_sc as plsc

_LANES = 128


@functools.cache
def _gather_call(B, E, dtype):
    info = plsc.get_sparse_core_info()
    NC, NS, L = info.num_cores, info.num_subcores, info.num_lanes
    NW = NC * NS
    half = NW // 2
    b_per_w = B // half
    n_pairs = b_per_w // L
    mesh = plsc.VectorSubcoreMesh(core_axis_name="c", subcore_axis_name="s")

    @functools.partial(
        pl.kernel,
        mesh=mesh,
        out_type=[
            jax.ShapeDtypeStruct((E, B), dtype),
            jax.ShapeDtypeStruct((E, B), dtype),
        ],
        scratch_types=[
            pltpu.VMEM((b_per_w,), jnp.int32),
            pltpu.VMEM((L // 2, E, _LANES), dtype),
            pltpu.VMEM((L // 2, E, _LANES), dtype),
            pltpu.VMEM((E, b_per_w), dtype),
            pltpu.SemaphoreType.DMA,
            pltpu.SemaphoreType.DMA,
        ],
        compiler_params=pltpu.CompilerParams(needs_layout_passes=False),
    )
    def gather_k(uids_hbm, iids_hbm, utabT_hbm, itabT_hbm, uT_out, vT_out,
                 idx_v, buf_a, buf_b, outT_v, sem_a, sem_b):
        wid = lax.axis_index("s") * NC + lax.axis_index("c")
        base = lax.rem(wid, half) * b_per_w
        e_lo = jnp.arange(L, dtype=jnp.int32)
        e_hi = e_lo + L
        HALF = L // 2

        def do_table(ids_hbm, tabT_hbm, out_hbm):
            pltpu.sync_copy(ids_hbm.at[pl.ds(base, b_per_w)], idx_v)

            def fire(buf, sem, idxvec, lane0):
                for j in range(HALF):
                    tc = idxvec[lane0 + j] // _LANES
                    off = pl.multiple_of(tc * _LANES, _LANES)
                    pltpu.async_copy(
                        tabT_hbm.at[:, pl.ds(off, _LANES)], buf.at[j], sem)

            def drain(buf, sem):
                for j in range(HALF):
                    pltpu.make_async_copy(
                        tabT_hbm.at[:, pl.ds(0, _LANES)],
                        buf.at[j], sem).wait()

            def extract(buf, idxvec, lane0, kbase):
                for j in range(HALF):
                    ln = jnp.full((L,), lax.rem(idxvec[lane0 + j], _LANES),
                                  dtype=jnp.int32)
                    k = jnp.full((L,), kbase + j, dtype=jnp.int32)
                    lo = plsc.load_gather(buf.at[j], [e_lo, ln])
                    hi = plsc.load_gather(buf.at[j], [e_hi, ln])
                    plsc.store_scatter(outT_v, [e_lo, k], lo)
                    plsc.store_scatter(outT_v, [e_hi, k], hi)

            idx0 = idx_v[pl.ds(0, L)]
            fire(buf_a, sem_a, idx0, 0)

            def pair(p, carry):
                idxvec = idx_v[pl.ds(p * L, L)]
                fire(buf_b, sem_b, idxvec, HALF)
                drain(buf_a, sem_a)
                extract(buf_a, idxvec, 0, p * L)

                @pl.when(p < n_pairs - 1)
                def _():
                    idxnext = idx_v[pl.ds((p + 1) * L, L)]
                    fire(buf_a, sem_a, idxnext, 0)

                drain(buf_b, sem_b)
                extract(buf_b, idxvec, HALF, p * L + HALF)
                return carry

            lax.fori_loop(0, n_pairs, pair, 0)
            pltpu.sync_copy(outT_v, out_hbm.at[:, pl.ds(base, b_per_w)])

        @pl.when(wid < half)
        def _():
            do_table(uids_hbm, utabT_hbm, uT_out)

        @pl.when(wid >= half)
        def _():
            do_table(iids_hbm, itabT_hbm, vT_out)

    return gather_k


def _mlp_body(uT_ref, vT_ref, w1u_ref, w1v_ref, b1_ref, w2_ref, b2_ref,
              out_ref):
    hT = jnp.dot(w1u_ref[...], uT_ref[...], preferred_element_type=jnp.float32)
    hT = hT + jnp.dot(w1v_ref[...], vT_ref[...],
                      preferred_element_type=jnp.float32)
    hT = jnp.maximum(hT + b1_ref[...], 0.0)
    out_ref[...] = (
        jnp.dot(w2_ref[...], hT, preferred_element_type=jnp.float32)
        + b2_ref[0, 0]
    )


@functools.cache
def _mlp_call(B, E, H, BB):
    grid = (B // BB,)
    return pl.pallas_call(
        _mlp_body,
        grid=grid,
        in_specs=[
            pl.BlockSpec((E, BB), lambda i: (0, i)),
            pl.BlockSpec((E, BB), lambda i: (0, i)),
            pl.BlockSpec((H, E), lambda i: (0, 0)),
            pl.BlockSpec((H, E), lambda i: (0, 0)),
            pl.BlockSpec((H, 1), lambda i: (0, 0)),
            pl.BlockSpec((1, H), lambda i: (0, 0)),
            pl.BlockSpec((1, 1), lambda i: (0, 0)),
        ],
        out_specs=pl.BlockSpec((1, BB), lambda i: (0, i)),
        out_shape=jax.ShapeDtypeStruct((1, B), jnp.float32),
    )


def kernel(user_ids, item_ids, user_table, item_table, W1, b1, W2, b2):
    B = user_ids.shape[0]
    E = user_table.shape[1]
    H = W1.shape[0]

    uT, vT = _gather_call(B, E, user_table.dtype)(
        user_ids, item_ids, user_table.T, item_table.T)

    w1u = W1[:, :E]
    w1v = W1[:, E:]
    logitsT = _mlp_call(B, E, H, 2048)(
        uT, vT, w1u, w1v, b1.reshape(H, 1), W2, b2.reshape(1, 1))
    return logitsT.reshape(B, 1)
